# baseline (speedup 1.0000x reference)
import jax
import jax.numpy as jnp
from jax import lax
from jax.experimental import pallas as pl
from jax.experimental.pallas import tpu as pltpu
from jax.experimental.pallas import tpu_sc as plsc

SEQ_LEN = 200
DIM = 128
BATCH = 4096
LANES = 16

NUM_CORES = 2
NUM_SUBCORES = 16
NUM_WORKERS = NUM_CORES * NUM_SUBCORES

CHUNK = 128
TOKENS = BATCH * SEQ_LEN
ROWS_PER_W = TOKENS // NUM_WORKERS
CHUNKS_PER_W = ROWS_PER_W // CHUNK
VECS_PER_ROW = DIM // LANES


def _body(idx_hbm, table_hbm, pos_hbm, out_hbm, idx_v, pos_v, buf0, buf1,
          buf2, gsem0, gsem1, gsem2, ssem0, ssem1, ssem2):
  c = lax.axis_index("c")
  s = lax.axis_index("s")
  wid = s * NUM_CORES + c

  pltpu.sync_copy(pos_hbm, pos_v.at[pl.ds(0, SEQ_LEN)])
  pltpu.sync_copy(pos_hbm, pos_v.at[pl.ds(SEQ_LEN, SEQ_LEN)])
  pltpu.sync_copy(idx_hbm.at[pl.ds(wid * CHUNKS_PER_W, CHUNKS_PER_W)], idx_v)

  row0 = wid * ROWS_PER_W
  bufs = (buf0, buf1, buf2)
  gsems = (gsem0, gsem1, gsem2)
  ssems = (ssem0, ssem1, ssem2)
  NBUF = 3

  def out_slice(gg):
    return out_hbm.at[pl.ds(row0 + gg * CHUNK, CHUNK)]

  def prefill(gg, buf):
    pbase = lax.rem(gg * CHUNK, SEQ_LEN)

    @plsc.parallel_loop(0, CHUNK, 1, unroll=4)
    def _copy_row(r):
      pr = pbase + r
      for v in range(VECS_PER_ROW):
        sl = pl.ds(v * LANES, LANES)
        buf[r, sl] = pos_v[pr, sl]

  HALF = CHUNK // 2

  def launch_gather(gg, buf, sem):
    pltpu.async_copy(table_hbm.at[idx_v.at[gg, pl.ds(0, HALF)]],
                     buf.at[pl.ds(0, HALF)], sem, add=True)
    pltpu.async_copy(table_hbm.at[idx_v.at[gg, pl.ds(HALF, HALF)]],
                     buf.at[pl.ds(HALF, HALF)], sem, add=True)

  prefill(0, buf0)
  launch_gather(0, buf0, gsem0)
  prefill(1, buf1)
  launch_gather(1, buf1, gsem1)

  def process(gg, p):
    buf = bufs[p]
    q = (p + 2) % NBUF
    @pl.when(gg >= 1)
    def _drain():
      pltpu.make_async_copy(bufs[q], out_slice(gg - 1), ssems[q]).wait()

    @pl.when(gg + 2 < CHUNKS_PER_W)
    def _start():
      prefill(gg + 2, bufs[q])
      launch_gather(gg + 2, bufs[q], gsems[q])

    pltpu.make_async_copy(table_hbm.at[idx_v.at[gg]], buf, gsems[p]).wait()
    pltpu.async_copy(buf, out_slice(gg), ssems[p])

  def outer(g, _):
    for p in range(NBUF):
      process(g * NBUF + p, p)
    return _

  n_main = CHUNKS_PER_W // NBUF * NBUF
  lax.fori_loop(0, CHUNKS_PER_W // NBUF, outer, None)

  for gg in range(n_main, CHUNKS_PER_W):
    process(gg, gg % NBUF)
  last = CHUNKS_PER_W - 1
  pltpu.make_async_copy(bufs[last % NBUF], out_slice(last),
                        ssems[last % NBUF]).wait()


@jax.jit
def _run(idx2d, token_table, pos_table):
  mesh = plsc.VectorSubcoreMesh(core_axis_name="c", subcore_axis_name="s")
  f = pl.kernel(
      _body,
      out_type=jax.ShapeDtypeStruct((TOKENS, DIM), jnp.float32),
      mesh=mesh,
      scratch_types=[
          pltpu.VMEM((CHUNKS_PER_W, CHUNK), jnp.int32),
          pltpu.VMEM((2 * SEQ_LEN, DIM), jnp.float32),
          pltpu.VMEM((CHUNK, DIM), jnp.float32),
          pltpu.VMEM((CHUNK, DIM), jnp.float32),
          pltpu.VMEM((CHUNK, DIM), jnp.float32),
          pltpu.SemaphoreType.DMA,
          pltpu.SemaphoreType.DMA,
          pltpu.SemaphoreType.DMA,
          pltpu.SemaphoreType.DMA,
          pltpu.SemaphoreType.DMA,
          pltpu.SemaphoreType.DMA,
      ],
  )
  return f(idx2d, token_table, pos_table)


def kernel(inputs, token_table, pos_table):
  idx2d = inputs.astype(jnp.int32).reshape(TOKENS // CHUNK, CHUNK)
  out = _run(idx2d, token_table, pos_table)
  return out.reshape(BATCH, SEQ_LEN, DIM)


# speedup vs baseline: 1.0022x; 1.0022x over previous
import math

import jax
import jax.numpy as jnp
from jax import lax
from jax.experimental import pallas as pl
from jax.experimental.pallas import tpu as pltpu
from jax.experimental.pallas import tpu_sc as plsc

SEQ_LEN = 200
DIM = 128
BATCH = 4096
LANES = 16

NUM_CORES = 2
NUM_SUBCORES = 16
NUM_WORKERS = NUM_CORES * NUM_SUBCORES

CHUNK = 64
NBUF = 5
TOKENS = BATCH * SEQ_LEN
ROWS_PER_W = TOKENS // NUM_WORKERS
CHUNKS_PER_W = ROWS_PER_W // CHUNK
VECS_PER_ROW = DIM // LANES
AHEAD = NBUF - 1

_MAX_PBASE = SEQ_LEN - math.gcd(CHUNK, SEQ_LEN)
POS_ROWS = _MAX_PBASE + CHUNK
_EXTRA = POS_ROWS - SEQ_LEN


def _body(idx_hbm, table_hbm, pos_hbm, out_hbm, idx_v, pos_v, *rest):
  bufs = rest[:NBUF]
  gsems = rest[NBUF:2 * NBUF]
  ssems = rest[2 * NBUF:3 * NBUF]

  c = lax.axis_index("c")
  s = lax.axis_index("s")
  wid = s * NUM_CORES + c

  pltpu.sync_copy(pos_hbm, pos_v.at[pl.ds(0, SEQ_LEN)])
  if _EXTRA:
    pltpu.sync_copy(pos_hbm.at[pl.ds(0, _EXTRA)],
                    pos_v.at[pl.ds(SEQ_LEN, _EXTRA)])
  pltpu.sync_copy(idx_hbm.at[pl.ds(wid * CHUNKS_PER_W, CHUNKS_PER_W)], idx_v)

  row0 = wid * ROWS_PER_W

  def out_slice(gg):
    return out_hbm.at[pl.ds(row0 + gg * CHUNK, CHUNK)]

  def prefill(gg, buf):
    pbase = lax.rem(gg * CHUNK, SEQ_LEN)

    @plsc.parallel_loop(0, CHUNK, 1, unroll=4)
    def _copy_row(r):
      pr = pbase + r
      for v in range(VECS_PER_ROW):
        sl = pl.ds(v * LANES, LANES)
        buf[r, sl] = pos_v[pr, sl]

  def launch_gather(gg, buf, sem):
    pltpu.async_copy(table_hbm.at[idx_v.at[gg]], buf, sem, add=True)

  for k in range(AHEAD):
    prefill(k, bufs[k])
    launch_gather(k, bufs[k], gsems[k])

  def process(gg, p):
    buf = bufs[p]
    q = (p + AHEAD) % NBUF
    @pl.when(gg >= 1)
    def _drain():
      pltpu.make_async_copy(bufs[q], out_slice(gg - 1), ssems[q]).wait()

    @pl.when(gg + AHEAD < CHUNKS_PER_W)
    def _start():
      prefill(gg + AHEAD, bufs[q])
      launch_gather(gg + AHEAD, bufs[q], gsems[q])

    pltpu.make_async_copy(table_hbm.at[idx_v.at[gg]], buf, gsems[p]).wait()
    pltpu.async_copy(buf, out_slice(gg), ssems[p])

  def outer(g, _):
    for p in range(NBUF):
      process(g * NBUF + p, p)
    return _

  n_main = CHUNKS_PER_W // NBUF * NBUF
  lax.fori_loop(0, CHUNKS_PER_W // NBUF, outer, None)

  for gg in range(n_main, CHUNKS_PER_W):
    process(gg, gg % NBUF)
  last = CHUNKS_PER_W - 1
  pltpu.make_async_copy(bufs[last % NBUF], out_slice(last),
                        ssems[last % NBUF]).wait()


@jax.jit
def _run(idx2d, token_table, pos_table):
  mesh = plsc.VectorSubcoreMesh(core_axis_name="c", subcore_axis_name="s")
  f = pl.kernel(
      _body,
      out_type=jax.ShapeDtypeStruct((TOKENS, DIM), jnp.float32),
      mesh=mesh,
      scratch_types=(
          [pltpu.VMEM((CHUNKS_PER_W, CHUNK), jnp.int32),
           pltpu.VMEM((POS_ROWS, DIM), jnp.float32)]
          + [pltpu.VMEM((CHUNK, DIM), jnp.float32)] * NBUF
          + [pltpu.SemaphoreType.DMA] * (2 * NBUF)
      ),
  )
  return f(idx2d, token_table, pos_table)


def kernel(inputs, token_table, pos_table):
  idx2d = inputs.astype(jnp.int32).reshape(TOKENS // CHUNK, CHUNK)
  out = _run(idx2d, token_table, pos_table)
  return out.reshape(BATCH, SEQ_LEN, DIM)


# speedup vs baseline: 1.0113x; 1.0091x over previous
import math

import jax
import jax.numpy as jnp
from jax import lax
from jax.experimental import pallas as pl
from jax.experimental.pallas import tpu as pltpu
from jax.experimental.pallas import tpu_sc as plsc

SEQ_LEN = 200
DIM = 128
BATCH = 4096
LANES = 16

NUM_CORES = 2
NUM_SUBCORES = 16
NUM_WORKERS = NUM_CORES * NUM_SUBCORES

CHUNK = 128
NBUF = 3
TOKENS = BATCH * SEQ_LEN
ROWS_PER_W = TOKENS // NUM_WORKERS
CHUNKS_PER_W = ROWS_PER_W // CHUNK
VECS_PER_ROW = DIM // LANES
AHEAD = NBUF - 1

_MAX_PBASE = SEQ_LEN - math.gcd(CHUNK, SEQ_LEN)
POS_ROWS = _MAX_PBASE + CHUNK
_EXTRA = POS_ROWS - SEQ_LEN


def _body(idx_hbm, table_hbm, pos_hbm, out_hbm, idx_v, pos_v, *rest):
  bufs = rest[:NBUF]
  gsems = rest[NBUF:2 * NBUF]
  ssems = rest[2 * NBUF:3 * NBUF]

  c = lax.axis_index("c")
  s = lax.axis_index("s")
  wid = s * NUM_CORES + c

  pltpu.sync_copy(pos_hbm, pos_v.at[pl.ds(0, SEQ_LEN)])
  if _EXTRA:
    pltpu.sync_copy(pos_hbm.at[pl.ds(0, _EXTRA)],
                    pos_v.at[pl.ds(SEQ_LEN, _EXTRA)])
  pltpu.sync_copy(idx_hbm.at[pl.ds(wid * CHUNKS_PER_W, CHUNKS_PER_W)], idx_v)

  row0 = wid * ROWS_PER_W

  def out_slice(gg):
    return out_hbm.at[pl.ds(row0 + gg * CHUNK, CHUNK)]

  def prefill(gg, buf):
    pbase = lax.rem(gg * CHUNK, SEQ_LEN)

    @plsc.parallel_loop(0, CHUNK, 1, unroll=4)
    def _copy_row(r):
      pr = pbase + r
      for v in range(VECS_PER_ROW):
        sl = pl.ds(v * LANES, LANES)
        buf[r, sl] = pos_v[pr, sl]

  def launch_gather(gg, buf, sem):
    pltpu.async_copy(table_hbm.at[idx_v.at[gg]], buf, sem, add=True)

  for k in range(AHEAD):
    prefill(k, bufs[k])
    launch_gather(k, bufs[k], gsems[k])

  def process(gg, p):
    buf = bufs[p]
    q = (p + AHEAD) % NBUF
    pltpu.make_async_copy(table_hbm.at[idx_v.at[gg]], buf, gsems[p]).wait()

    @pl.when(gg >= 1)
    def _drain():
      pltpu.make_async_copy(bufs[q], out_slice(gg - 1), ssems[q]).wait()

    @pl.when(gg + AHEAD < CHUNKS_PER_W)
    def _start():
      prefill(gg + AHEAD, bufs[q])
      launch_gather(gg + AHEAD, bufs[q], gsems[q])

    pltpu.async_copy(buf, out_slice(gg), ssems[p])

  def outer(g, _):
    for p in range(NBUF):
      process(g * NBUF + p, p)
    return _

  n_main = CHUNKS_PER_W // NBUF * NBUF
  lax.fori_loop(0, CHUNKS_PER_W // NBUF, outer, None)

  for gg in range(n_main, CHUNKS_PER_W):
    process(gg, gg % NBUF)
  last = CHUNKS_PER_W - 1
  pltpu.make_async_copy(bufs[last % NBUF], out_slice(last),
                        ssems[last % NBUF]).wait()


@jax.jit
def _run(idx2d, token_table, pos_table):
  mesh = plsc.VectorSubcoreMesh(core_axis_name="c", subcore_axis_name="s")
  f = pl.kernel(
      _body,
      out_type=jax.ShapeDtypeStruct((TOKENS, DIM), jnp.float32),
      mesh=mesh,
      scratch_types=(
          [pltpu.VMEM((CHUNKS_PER_W, CHUNK), jnp.int32),
           pltpu.VMEM((POS_ROWS, DIM), jnp.float32)]
          + [pltpu.VMEM((CHUNK, DIM), jnp.float32)] * NBUF
          + [pltpu.SemaphoreType.DMA] * (2 * NBUF)
      ),
  )
  return f(idx2d, token_table, pos_table)


def kernel(inputs, token_table, pos_table):
  idx2d = inputs.astype(jnp.int32).reshape(TOKENS // CHUNK, CHUNK)
  out = _run(idx2d, token_table, pos_table)
  return out.reshape(BATCH, SEQ_LEN, DIM)


# speedup vs baseline: 1.0189x; 1.0075x over previous
import math

import jax
import jax.numpy as jnp
from jax import lax
from jax.experimental import pallas as pl
from jax.experimental.pallas import tpu as pltpu
from jax.experimental.pallas import tpu_sc as plsc

SEQ_LEN = 200
DIM = 128
BATCH = 4096
LANES = 16

NUM_CORES = 2
NUM_SUBCORES = 16
NUM_WORKERS = NUM_CORES * NUM_SUBCORES

CHUNK = 80
NBUF = 5
STORE_DEPTH = 2
TOKENS = BATCH * SEQ_LEN
ROWS_PER_W = TOKENS // NUM_WORKERS
CHUNKS_PER_W = ROWS_PER_W // CHUNK
VECS_PER_ROW = DIM // LANES
AHEAD = NBUF - STORE_DEPTH

_MAX_PBASE = SEQ_LEN - math.gcd(CHUNK, SEQ_LEN)
POS_ROWS = _MAX_PBASE + CHUNK
_EXTRA = POS_ROWS - SEQ_LEN


def _body(idx_hbm, table_hbm, pos_hbm, out_hbm, idx_v, pos_v, *rest):
  bufs = rest[:NBUF]
  gsems = rest[NBUF:2 * NBUF]
  ssems = rest[2 * NBUF:3 * NBUF]

  c = lax.axis_index("c")
  s = lax.axis_index("s")
  wid = s * NUM_CORES + c

  pltpu.sync_copy(pos_hbm, pos_v.at[pl.ds(0, SEQ_LEN)])
  if _EXTRA:
    pltpu.sync_copy(pos_hbm.at[pl.ds(0, _EXTRA)],
                    pos_v.at[pl.ds(SEQ_LEN, _EXTRA)])
  pltpu.sync_copy(idx_hbm.at[pl.ds(wid * CHUNKS_PER_W, CHUNKS_PER_W)], idx_v)

  row0 = wid * ROWS_PER_W

  def out_slice(gg):
    return out_hbm.at[pl.ds(row0 + gg * CHUNK, CHUNK)]

  def prefill(gg, buf):
    pbase = lax.rem(gg * CHUNK, SEQ_LEN)

    @plsc.parallel_loop(0, CHUNK, 1, unroll=4)
    def _copy_row(r):
      pr = pbase + r
      for v in range(VECS_PER_ROW):
        sl = pl.ds(v * LANES, LANES)
        buf[r, sl] = pos_v[pr, sl]

  def launch_gather(gg, buf, sem):
    pltpu.async_copy(table_hbm.at[idx_v.at[gg]], buf, sem, add=True)

  for k in range(AHEAD):
    prefill(k, bufs[k])
    launch_gather(k, bufs[k], gsems[k])

  def process(gg, p):
    buf = bufs[p]
    q = (p + AHEAD) % NBUF
    pltpu.make_async_copy(table_hbm.at[idx_v.at[gg]], buf, gsems[p]).wait()

    @pl.when(gg >= STORE_DEPTH)
    def _drain():
      pltpu.make_async_copy(bufs[q], out_slice(gg - STORE_DEPTH),
                            ssems[q]).wait()

    @pl.when(gg + AHEAD < CHUNKS_PER_W)
    def _start():
      prefill(gg + AHEAD, bufs[q])
      launch_gather(gg + AHEAD, bufs[q], gsems[q])

    pltpu.async_copy(buf, out_slice(gg), ssems[p])

  def outer(g, _):
    for p in range(NBUF):
      process(g * NBUF + p, p)
    return _

  n_main = CHUNKS_PER_W // NBUF * NBUF
  lax.fori_loop(0, CHUNKS_PER_W // NBUF, outer, None)

  for gg in range(n_main, CHUNKS_PER_W):
    process(gg, gg % NBUF)
  for gg in range(CHUNKS_PER_W - STORE_DEPTH, CHUNKS_PER_W):
    pltpu.make_async_copy(bufs[gg % NBUF], out_slice(gg),
                          ssems[gg % NBUF]).wait()


@jax.jit
def _run(idx2d, token_table, pos_table):
  mesh = plsc.VectorSubcoreMesh(core_axis_name="c", subcore_axis_name="s")
  f = pl.kernel(
      _body,
      out_type=jax.ShapeDtypeStruct((TOKENS, DIM), jnp.float32),
      mesh=mesh,
      scratch_types=(
          [pltpu.VMEM((CHUNKS_PER_W, CHUNK), jnp.int32),
           pltpu.VMEM((POS_ROWS, DIM), jnp.float32)]
          + [pltpu.VMEM((CHUNK, DIM), jnp.float32)] * NBUF
          + [pltpu.SemaphoreType.DMA] * (2 * NBUF)
      ),
  )
  return f(idx2d, token_table, pos_table)


def kernel(inputs, token_table, pos_table):
  idx2d = inputs.astype(jnp.int32).reshape(TOKENS // CHUNK, CHUNK)
  out = _run(idx2d, token_table, pos_table)
  return out.reshape(BATCH, SEQ_LEN, DIM)
